# baseline (speedup 1.0000x reference)
import functools

import jax
import jax.numpy as jnp
from jax import lax
from jax.experimental import pallas as pl
from jax.experimental.pallas import tpu as pltpu
from jax.experimental.pallas import tpu_sc as plsc

N = 10000
E = 320000
D = 128
DE = 16

NC, NS, L = 2, 16, 16
NW = NC * NS
EPT = E // NW
K = 40
NCH = EPT // K
NPAD = 10112
RPT = NPAD // NS


def _ab_body(x_ref, w1_ref, w2_ref, be_ref, a_ref, b_ref):
    xb = x_ref[...]
    a_ref[...] = jnp.dot(xb, w1_ref[...], preferred_element_type=jnp.float32)
    b_ref[...] = (jnp.dot(xb, w2_ref[...], preferred_element_type=jnp.float32)
                  + be_ref[...])


def _compute_ab(x, w1, w2, be):
    BR = 1000
    return pl.pallas_call(
        _ab_body,
        grid=(N // BR,),
        in_specs=[
            pl.BlockSpec((BR, D), lambda i: (i, 0)),
            pl.BlockSpec((D, D), lambda i: (0, 0)),
            pl.BlockSpec((D, D), lambda i: (0, 0)),
            pl.BlockSpec((1, D), lambda i: (0, 0)),
        ],
        out_specs=[pl.BlockSpec((BR, D), lambda i: (i, 0)),
                   pl.BlockSpec((BR, D), lambda i: (i, 0))],
        out_shape=[jax.ShapeDtypeStruct((N, D), jnp.float32),
                   jax.ShapeDtypeStruct((N, D), jnp.float32)],
    )(x, w1, w2, be)


def _c_body(ea_ref, w3_ref, c_ref):
    c_ref[...] = jnp.dot(ea_ref[...], w3_ref[...],
                         preferred_element_type=jnp.float32)


def _compute_c(ea, w3):
    BR = 4000
    return pl.pallas_call(
        _c_body,
        grid=(E // BR,),
        in_specs=[pl.BlockSpec((BR, DE), lambda i: (i, 0)),
                  pl.BlockSpec((DE, D), lambda i: (0, 0))],
        out_specs=pl.BlockSpec((BR, D), lambda i: (i, 0)),
        out_shape=jax.ShapeDtypeStruct((E, D), jnp.float32),
    )(ea, w3)


def _sc_edge(src, dst, a_t, b_t, c_t, zeros):
    mesh = plsc.VectorSubcoreMesh(core_axis_name="c", subcore_axis_name="s")
    DEPTH = 3

    @functools.partial(
        pl.kernel,
        mesh=mesh,
        out_type=jax.ShapeDtypeStruct((NC, NPAD, D), jnp.float32),
        scratch_types=(
            [pltpu.VMEM((K,), jnp.int32) for _ in range(2 * DEPTH)]
            + [pltpu.VMEM((K, D), jnp.float32) for _ in range(3 * DEPTH)]
            + [pltpu.VMEM_SHARED((NPAD, D), jnp.float32)]
            + [pltpu.SemaphoreType.DMA for _ in range(2 * DEPTH + 1)]
        ),
    )
    def run(src_h, dst_h, a_h, b_h, c_h, z_h, out_h, *refs):
        sidx = refs[0:DEPTH]
        didx = refs[DEPTH:2 * DEPTH]
        av = refs[2 * DEPTH:3 * DEPTH]
        bv = refs[3 * DEPTH:4 * DEPTH]
        cv = refs[4 * DEPTH:5 * DEPTH]
        agg = refs[5 * DEPTH]
        isem = refs[5 * DEPTH + 1:6 * DEPTH + 1]
        gsem = refs[6 * DEPTH + 1:7 * DEPTH + 1]
        ssem = refs[7 * DEPTH + 1]
        cid = lax.axis_index("c")
        sid = lax.axis_index("s")
        wid = cid * NS + sid
        pltpu.sync_copy(z_h, agg.at[pl.ds(sid * RPT, RPT)])
        plsc.subcore_barrier()
        ebase = wid * EPT

        def fetch_idx(n, b):
            off = ebase + n * K
            pltpu.async_copy(src_h.at[pl.ds(off, K)], sidx[b], isem[b])
            pltpu.async_copy(dst_h.at[pl.ds(off, K)], didx[b], isem[b])

        def wait_idx(n, b):
            off = ebase + n * K
            pltpu.make_async_copy(src_h.at[pl.ds(off, K)], sidx[b],
                                  isem[b]).wait()
            pltpu.make_async_copy(dst_h.at[pl.ds(off, K)], didx[b],
                                  isem[b]).wait()

        def fetch_rows(n, b):
            pltpu.async_copy(a_h.at[sidx[b]], av[b], gsem[b])
            pltpu.async_copy(b_h.at[didx[b]], bv[b], gsem[b])
            pltpu.async_copy(c_h.at[pl.ds(ebase + n * K, K)], cv[b], gsem[b])

        def wait_rows(n, b):
            pltpu.make_async_copy(a_h.at[sidx[b]], av[b], gsem[b]).wait()
            pltpu.make_async_copy(b_h.at[didx[b]], bv[b], gsem[b]).wait()
            pltpu.make_async_copy(c_h.at[pl.ds(ebase + n * K, K)], cv[b],
                                  gsem[b]).wait()

        def compute(b):
            @plsc.parallel_loop(0, K, unroll=2)
            def row(r):
                for j in range(8):
                    sl = pl.ds(j * L, L)
                    t = av[b][r, sl] + bv[b][r, sl] + cv[b][r, sl]
                    cv[b][r, sl] = t / (1.0 + jnp.exp(-t))

        def body(g, carry):
            base = DEPTH * g
            for b in range(DEPTH):
                fetch_idx(base + b, b)
            for b in range(DEPTH):
                wait_idx(base + b, b)
                fetch_rows(base + b, b)
            for b in range(DEPTH):
                wait_rows(base + b, b)
                compute(b)
                pltpu.async_copy(cv[b], agg.at[didx[b]], ssem, add=True)
            for b in range(DEPTH):
                pltpu.make_async_copy(cv[b], agg.at[didx[b]], ssem).wait()
            return carry

        lax.fori_loop(0, NCH // DEPTH, body, 0)
        nlast = NCH - 1
        fetch_idx(nlast, 0)
        wait_idx(nlast, 0)
        fetch_rows(nlast, 0)
        wait_rows(nlast, 0)
        compute(0)
        pltpu.sync_copy(cv[0], agg.at[didx[0]], add=True)
        plsc.subcore_barrier()
        pltpu.sync_copy(agg.at[pl.ds(sid * RPT, RPT)],
                        out_h.at[cid, pl.ds(sid * RPT, RPT)])

    return run(src, dst, a_t, b_t, c_t, zeros)


def _node_body(x_ref, p_ref, w1_ref, w2_ref, bn_ref, o_ref):
    xb = x_ref[...]
    agg = p_ref[0] + p_ref[1]
    h = (jnp.dot(xb, w1_ref[...], preferred_element_type=jnp.float32)
         + jnp.dot(agg, w2_ref[...], preferred_element_type=jnp.float32)
         + bn_ref[...])
    o_ref[...] = xb + h / (1.0 + jnp.exp(-h))


def _node_update(x, partials, wn1, wn2, bn):
    BR = 1000
    return pl.pallas_call(
        _node_body,
        grid=(N // BR,),
        in_specs=[
            pl.BlockSpec((BR, D), lambda i: (i, 0)),
            pl.BlockSpec((NC, BR, D), lambda i: (0, i, 0)),
            pl.BlockSpec((D, D), lambda i: (0, 0)),
            pl.BlockSpec((D, D), lambda i: (0, 0)),
            pl.BlockSpec((1, D), lambda i: (0, 0)),
        ],
        out_specs=pl.BlockSpec((BR, D), lambda i: (i, 0)),
        out_shape=jax.ShapeDtypeStruct((N, D), jnp.float32),
    )(x, partials, wn1, wn2, bn)


def kernel(x, edge_index, edge_attr, W_edge, b_edge, W_node, b_node):
    src = edge_index[0].astype(jnp.int32)
    dst = edge_index[1].astype(jnp.int32)
    w1 = W_edge[:D]
    w2 = W_edge[D:2 * D]
    w3 = W_edge[2 * D:]
    a_t, b_t = _compute_ab(x, w1, w2, b_edge.reshape(1, D))
    c_t = _compute_c(edge_attr, w3)
    zeros = jnp.zeros((RPT, D), jnp.float32)
    partials = _sc_edge(src, dst, a_t, b_t, c_t, zeros)
    return _node_update(x, partials, W_node[:D], W_node[D:], b_node.reshape(1, D))


# speedup vs baseline: 1.2994x; 1.2994x over previous
import functools

import jax
import jax.numpy as jnp
from jax import lax
from jax.experimental import pallas as pl
from jax.experimental.pallas import tpu as pltpu
from jax.experimental.pallas import tpu_sc as plsc

N = 10000
E = 320000
D = 128
DE = 16

NC, NS, L = 2, 16, 16
NW = NC * NS
EPT = E // NW
K = 40
NCH = EPT // K
NPAD = 10112
RPT = NPAD // NS


def _ab_body(x_ref, w1_ref, w2_ref, be_ref, a_ref, b_ref):
    xb = x_ref[...]
    a_ref[...] = jnp.dot(xb, w1_ref[...], preferred_element_type=jnp.float32)
    b_ref[...] = (jnp.dot(xb, w2_ref[...], preferred_element_type=jnp.float32)
                  + be_ref[...])


def _compute_ab(x, w1, w2, be):
    BR = 1000
    return pl.pallas_call(
        _ab_body,
        grid=(N // BR,),
        in_specs=[
            pl.BlockSpec((BR, D), lambda i: (i, 0)),
            pl.BlockSpec((D, D), lambda i: (0, 0)),
            pl.BlockSpec((D, D), lambda i: (0, 0)),
            pl.BlockSpec((1, D), lambda i: (0, 0)),
        ],
        out_specs=[pl.BlockSpec((BR, D), lambda i: (i, 0)),
                   pl.BlockSpec((BR, D), lambda i: (i, 0))],
        out_shape=[jax.ShapeDtypeStruct((N, D), jnp.float32),
                   jax.ShapeDtypeStruct((N, D), jnp.float32)],
    )(x, w1, w2, be)


def _c_body(eat_ref, w3_ref, c_ref):
    c_ref[...] = jax.lax.dot_general(
        eat_ref[...], w3_ref[...], (((0,), (0,)), ((), ())),
        preferred_element_type=jnp.float32)


def _compute_c(eat, w3):
    BR = 3200
    return pl.pallas_call(
        _c_body,
        grid=(E // BR,),
        in_specs=[pl.BlockSpec((DE, BR), lambda i: (0, i)),
                  pl.BlockSpec((DE, D), lambda i: (0, 0))],
        out_specs=pl.BlockSpec((BR, D), lambda i: (i, 0)),
        out_shape=jax.ShapeDtypeStruct((E, D), jnp.float32),
    )(eat, w3)


def _sc_edge(src, dst, a_t, b_t, c_t, zeros):
    mesh = plsc.VectorSubcoreMesh(core_axis_name="c", subcore_axis_name="s")
    DEPTH = 3

    @functools.partial(
        pl.kernel,
        mesh=mesh,
        out_type=jax.ShapeDtypeStruct((NC, NPAD, D), jnp.float32),
        scratch_types=(
            [pltpu.VMEM((K,), jnp.int32) for _ in range(2 * DEPTH)]
            + [pltpu.VMEM((K, D), jnp.float32) for _ in range(3 * DEPTH)]
            + [pltpu.VMEM_SHARED((NPAD, D), jnp.float32)]
            + [pltpu.SemaphoreType.DMA for _ in range(2 * DEPTH + 1)]
        ),
    )
    def run(src_h, dst_h, a_h, b_h, c_h, z_h, out_h, *refs):
        sidx = refs[0:DEPTH]
        didx = refs[DEPTH:2 * DEPTH]
        av = refs[2 * DEPTH:3 * DEPTH]
        bv = refs[3 * DEPTH:4 * DEPTH]
        cv = refs[4 * DEPTH:5 * DEPTH]
        agg = refs[5 * DEPTH]
        isem = refs[5 * DEPTH + 1:6 * DEPTH + 1]
        gsem = refs[6 * DEPTH + 1:7 * DEPTH + 1]
        ssem = refs[7 * DEPTH + 1]
        cid = lax.axis_index("c")
        sid = lax.axis_index("s")
        wid = cid * NS + sid
        pltpu.sync_copy(z_h, agg.at[pl.ds(sid * RPT, RPT)])
        plsc.subcore_barrier()
        ebase = wid * EPT

        def fetch_idx(n, b):
            off = ebase + n * K
            pltpu.async_copy(src_h.at[pl.ds(off, K)], sidx[b], isem[b])
            pltpu.async_copy(dst_h.at[pl.ds(off, K)], didx[b], isem[b])

        def wait_idx(n, b):
            off = ebase + n * K
            pltpu.make_async_copy(src_h.at[pl.ds(off, K)], sidx[b],
                                  isem[b]).wait()
            pltpu.make_async_copy(dst_h.at[pl.ds(off, K)], didx[b],
                                  isem[b]).wait()

        def fetch_rows(n, b):
            pltpu.async_copy(a_h.at[sidx[b]], av[b], gsem[b])
            pltpu.async_copy(b_h.at[didx[b]], bv[b], gsem[b])
            pltpu.async_copy(c_h.at[pl.ds(ebase + n * K, K)], cv[b], gsem[b])

        def wait_rows(n, b):
            pltpu.make_async_copy(a_h.at[sidx[b]], av[b], gsem[b]).wait()
            pltpu.make_async_copy(b_h.at[didx[b]], bv[b], gsem[b]).wait()
            pltpu.make_async_copy(c_h.at[pl.ds(ebase + n * K, K)], cv[b],
                                  gsem[b]).wait()

        def compute(b):
            def row(r, c2):
                for j in range(8):
                    sl = pl.ds(j * L, L)
                    t = av[b][r, sl] + bv[b][r, sl] + cv[b][r, sl]
                    cv[b][r, sl] = t / (1.0 + jnp.exp(-t))
                return c2
            lax.fori_loop(0, K, row, 0)

        def body(g, carry):
            base = DEPTH * g
            for b in range(DEPTH):
                fetch_idx(base + b, b)
            for b in range(DEPTH):
                wait_idx(base + b, b)
                fetch_rows(base + b, b)
            for b in range(DEPTH):
                wait_rows(base + b, b)
                compute(b)
                pltpu.async_copy(cv[b], agg.at[didx[b]], ssem, add=True)
            for b in range(DEPTH):
                pltpu.make_async_copy(cv[b], agg.at[didx[b]], ssem).wait()
            return carry

        lax.fori_loop(0, NCH // DEPTH, body, 0)
        nlast = NCH - 1
        fetch_idx(nlast, 0)
        wait_idx(nlast, 0)
        fetch_rows(nlast, 0)
        wait_rows(nlast, 0)
        compute(0)
        pltpu.sync_copy(cv[0], agg.at[didx[0]], add=True)
        plsc.subcore_barrier()
        pltpu.sync_copy(agg.at[pl.ds(sid * RPT, RPT)],
                        out_h.at[cid, pl.ds(sid * RPT, RPT)])

    return run(src, dst, a_t, b_t, c_t, zeros)


def _node_body(x_ref, p_ref, w1_ref, w2_ref, bn_ref, o_ref):
    xb = x_ref[...]
    agg = p_ref[0] + p_ref[1]
    h = (jnp.dot(xb, w1_ref[...], preferred_element_type=jnp.float32)
         + jnp.dot(agg, w2_ref[...], preferred_element_type=jnp.float32)
         + bn_ref[...])
    o_ref[...] = xb + h / (1.0 + jnp.exp(-h))


def _node_update(x, partials, wn1, wn2, bn):
    BR = 1000
    return pl.pallas_call(
        _node_body,
        grid=(N // BR,),
        in_specs=[
            pl.BlockSpec((BR, D), lambda i: (i, 0)),
            pl.BlockSpec((NC, BR, D), lambda i: (0, i, 0)),
            pl.BlockSpec((D, D), lambda i: (0, 0)),
            pl.BlockSpec((D, D), lambda i: (0, 0)),
            pl.BlockSpec((1, D), lambda i: (0, 0)),
        ],
        out_specs=pl.BlockSpec((BR, D), lambda i: (i, 0)),
        out_shape=jax.ShapeDtypeStruct((N, D), jnp.float32),
    )(x, partials, wn1, wn2, bn)


def kernel(x, edge_index, edge_attr, W_edge, b_edge, W_node, b_node):
    src = edge_index[0].astype(jnp.int32)
    dst = edge_index[1].astype(jnp.int32)
    w1 = W_edge[:D]
    w2 = W_edge[D:2 * D]
    w3 = W_edge[2 * D:]
    a_t, b_t = _compute_ab(x, w1, w2, b_edge.reshape(1, D))
    c_t = _compute_c(edge_attr.T, w3)
    zeros = jnp.zeros((RPT, D), jnp.float32)
    partials = _sc_edge(src, dst, a_t, b_t, c_t, zeros)
    return _node_update(x, partials, W_node[:D], W_node[D:], b_node.reshape(1, D))


# speedup vs baseline: 1.6009x; 1.2320x over previous
import functools

import jax
import jax.numpy as jnp
from jax import lax
from jax.experimental import pallas as pl
from jax.experimental.pallas import tpu as pltpu
from jax.experimental.pallas import tpu_sc as plsc

N = 10000
E = 320000
D = 128
DE = 16

NC, NS, L = 2, 16, 16
NW = NC * NS
EPT = E // NW
K = 40
NCH = EPT // K
NPAD = 10112
RPT = NPAD // NS


def _ab_body(x_ref, w1_ref, w2_ref, be_ref, a_ref, b_ref):
    xb = x_ref[...]
    a_ref[...] = jnp.dot(xb, w1_ref[...], preferred_element_type=jnp.float32)
    b_ref[...] = (jnp.dot(xb, w2_ref[...], preferred_element_type=jnp.float32)
                  + be_ref[...])


def _compute_ab(x, w1, w2, be):
    BR = 1000
    return pl.pallas_call(
        _ab_body,
        grid=(N // BR,),
        in_specs=[
            pl.BlockSpec((BR, D), lambda i: (i, 0)),
            pl.BlockSpec((D, D), lambda i: (0, 0)),
            pl.BlockSpec((D, D), lambda i: (0, 0)),
            pl.BlockSpec((1, D), lambda i: (0, 0)),
        ],
        out_specs=[pl.BlockSpec((BR, D), lambda i: (i, 0)),
                   pl.BlockSpec((BR, D), lambda i: (i, 0))],
        out_shape=[jax.ShapeDtypeStruct((N, D), jnp.float32),
                   jax.ShapeDtypeStruct((N, D), jnp.float32)],
    )(x, w1, w2, be)


def _c_body(eat_ref, w3_ref, c_ref):
    c_ref[...] = jax.lax.dot_general(
        eat_ref[...], w3_ref[...], (((0,), (0,)), ((), ())),
        preferred_element_type=jnp.float32)


def _compute_c(eat, w3):
    BR = 3200
    return pl.pallas_call(
        _c_body,
        grid=(E // BR,),
        in_specs=[pl.BlockSpec((DE, BR), lambda i: (0, i)),
                  pl.BlockSpec((DE, D), lambda i: (0, 0))],
        out_specs=pl.BlockSpec((BR, D), lambda i: (i, 0)),
        out_shape=jax.ShapeDtypeStruct((E, D), jnp.float32),
    )(eat, w3)


def _sc_edge(src, dst, a_t, b_t, c_t, zeros):
    mesh = plsc.VectorSubcoreMesh(core_axis_name="c", subcore_axis_name="s")
    SETS = 3
    ISLOTS = 6
    BODY = 10

    @functools.partial(
        pl.kernel,
        mesh=mesh,
        out_type=jax.ShapeDtypeStruct((NC, NPAD, D), jnp.float32),
        scratch_types=(
            [pltpu.VMEM((K,), jnp.int32) for _ in range(2 * ISLOTS)]
            + [pltpu.VMEM((K, D), jnp.float32) for _ in range(3 * SETS)]
            + [pltpu.VMEM_SHARED((NPAD, D), jnp.float32)]
            + [pltpu.SemaphoreType.DMA for _ in range(6)]
        ),
    )
    def run(src_h, dst_h, a_h, b_h, c_h, z_h, out_h, *refs):
        sidx = refs[0:ISLOTS]
        didx = refs[ISLOTS:2 * ISLOTS]
        av = refs[2 * ISLOTS:2 * ISLOTS + SETS]
        bv = refs[2 * ISLOTS + SETS:2 * ISLOTS + 2 * SETS]
        cv = refs[2 * ISLOTS + 2 * SETS:2 * ISLOTS + 3 * SETS]
        agg = refs[2 * ISLOTS + 3 * SETS]
        sems = refs[2 * ISLOTS + 3 * SETS + 1:]
        isem = sems[0:2]
        gsem = sems[2:5]
        ssem = sems[5]
        cid = lax.axis_index("c")
        sid = lax.axis_index("s")
        wid = cid * NS + sid
        pltpu.sync_copy(z_h, agg.at[pl.ds(sid * RPT, RPT)])
        plsc.subcore_barrier()
        ebase = wid * EPT

        def fetch_idx(n, q):
            off = ebase + n * K
            sl = q % ISLOTS
            pltpu.async_copy(src_h.at[pl.ds(off, K)], sidx[sl], isem[q % 2])
            pltpu.async_copy(dst_h.at[pl.ds(off, K)], didx[sl], isem[q % 2])

        def wait_idx(n, q):
            off = ebase + n * K
            sl = q % ISLOTS
            pltpu.make_async_copy(src_h.at[pl.ds(off, K)], sidx[sl],
                                  isem[q % 2]).wait()
            pltpu.make_async_copy(dst_h.at[pl.ds(off, K)], didx[sl],
                                  isem[q % 2]).wait()

        def fetch_rows(n, q):
            sl, st = q % ISLOTS, q % SETS
            pltpu.async_copy(a_h.at[sidx[sl]], av[st], gsem[q % 3])
            pltpu.async_copy(b_h.at[didx[sl]], bv[st], gsem[q % 3])
            pltpu.async_copy(c_h.at[pl.ds(ebase + n * K, K)], cv[st],
                             gsem[q % 3])

        def wait_rows(n, q):
            sl, st = q % ISLOTS, q % SETS
            pltpu.make_async_copy(a_h.at[sidx[sl]], av[st],
                                  gsem[q % 3]).wait()
            pltpu.make_async_copy(b_h.at[didx[sl]], bv[st],
                                  gsem[q % 3]).wait()
            pltpu.make_async_copy(c_h.at[pl.ds(ebase + n * K, K)], cv[st],
                                  gsem[q % 3]).wait()

        def compute(q):
            st = q % SETS

            def row(r, c2):
                for j in range(8):
                    sl = pl.ds(j * L, L)
                    t = av[st][r, sl] + bv[st][r, sl] + cv[st][r, sl]
                    cv[st][r, sl] = t / (1.0 + jnp.exp(-t))
                return c2
            lax.fori_loop(0, K, row, 0)

        def scat(q):
            pltpu.async_copy(cv[q % SETS], agg.at[didx[q % ISLOTS]], ssem,
                             add=True)

        def wait_scat(q):
            pltpu.make_async_copy(cv[q % SETS], agg.at[didx[q % ISLOTS]],
                                  ssem).wait()

        def body(g, carry):
            base = BODY * g
            fetch_idx(base + 0, 0)
            fetch_idx(base + 1, 1)
            wait_idx(base + 0, 0)
            fetch_rows(base + 0, 0)
            fetch_idx(base + 2, 2)
            wait_idx(base + 1, 1)
            fetch_rows(base + 1, 1)
            fetch_idx(base + 3, 3)
            wait_idx(base + 2, 2)
            fetch_rows(base + 2, 2)
            for q in range(BODY):
                wait_rows(base + q, q)
                compute(q)
                scat(q)
                if q + 4 < BODY:
                    fetch_idx(base + q + 4, q + 4)
                wait_scat(q)
                if q + SETS < BODY:
                    wait_idx(base + q + SETS, q + SETS)
                    fetch_rows(base + q + SETS, q + SETS)
            return carry

        lax.fori_loop(0, NCH // BODY, body, 0)
        plsc.subcore_barrier()
        pltpu.sync_copy(agg.at[pl.ds(sid * RPT, RPT)],
                        out_h.at[cid, pl.ds(sid * RPT, RPT)])

    return run(src, dst, a_t, b_t, c_t, zeros)


def _node_body(x_ref, p_ref, w1_ref, w2_ref, bn_ref, o_ref):
    xb = x_ref[...]
    agg = p_ref[0] + p_ref[1]
    h = (jnp.dot(xb, w1_ref[...], preferred_element_type=jnp.float32)
         + jnp.dot(agg, w2_ref[...], preferred_element_type=jnp.float32)
         + bn_ref[...])
    o_ref[...] = xb + h / (1.0 + jnp.exp(-h))


def _node_update(x, partials, wn1, wn2, bn):
    BR = 1000
    return pl.pallas_call(
        _node_body,
        grid=(N // BR,),
        in_specs=[
            pl.BlockSpec((BR, D), lambda i: (i, 0)),
            pl.BlockSpec((NC, BR, D), lambda i: (0, i, 0)),
            pl.BlockSpec((D, D), lambda i: (0, 0)),
            pl.BlockSpec((D, D), lambda i: (0, 0)),
            pl.BlockSpec((1, D), lambda i: (0, 0)),
        ],
        out_specs=pl.BlockSpec((BR, D), lambda i: (i, 0)),
        out_shape=jax.ShapeDtypeStruct((N, D), jnp.float32),
    )(x, partials, wn1, wn2, bn)


def kernel(x, edge_index, edge_attr, W_edge, b_edge, W_node, b_node):
    src = edge_index[0].astype(jnp.int32)
    dst = edge_index[1].astype(jnp.int32)
    w1 = W_edge[:D]
    w2 = W_edge[D:2 * D]
    w3 = W_edge[2 * D:]
    a_t, b_t = _compute_ab(x, w1, w2, b_edge.reshape(1, D))
    c_t = _compute_c(edge_attr.T, w3)
    zeros = jnp.zeros((RPT, D), jnp.float32)
    partials = _sc_edge(src, dst, a_t, b_t, c_t, zeros)
    return _node_update(x, partials, W_node[:D], W_node[D:], b_node.reshape(1, D))


# speedup vs baseline: 1.9302x; 1.2057x over previous
import functools

import jax
import jax.numpy as jnp
from jax import lax
from jax.experimental import pallas as pl
from jax.experimental.pallas import tpu as pltpu
from jax.experimental.pallas import tpu_sc as plsc

N = 10000
E = 320000
D = 128
DE = 16

NC, NS, L = 2, 16, 16
NW = NC * NS
EPT = E // NW
K = 40
NCH = EPT // K
NPAD = 10112
RPT = NPAD // NS


def _ab_body(x_ref, w1_ref, w2_ref, be_ref, a_ref, b_ref):
    xb = x_ref[...]
    a_ref[...] = jnp.dot(xb, w1_ref[...], preferred_element_type=jnp.float32)
    b_ref[...] = (jnp.dot(xb, w2_ref[...], preferred_element_type=jnp.float32)
                  + be_ref[...])


def _compute_ab(x, w1, w2, be):
    BR = 1000
    return pl.pallas_call(
        _ab_body,
        grid=(N // BR,),
        in_specs=[
            pl.BlockSpec((BR, D), lambda i: (i, 0)),
            pl.BlockSpec((D, D), lambda i: (0, 0)),
            pl.BlockSpec((D, D), lambda i: (0, 0)),
            pl.BlockSpec((1, D), lambda i: (0, 0)),
        ],
        out_specs=[pl.BlockSpec((BR, D), lambda i: (i, 0)),
                   pl.BlockSpec((BR, D), lambda i: (i, 0))],
        out_shape=[jax.ShapeDtypeStruct((N, D), jnp.float32),
                   jax.ShapeDtypeStruct((N, D), jnp.float32)],
    )(x, w1, w2, be)


def _c_body(eat_ref, w3_ref, c_ref):
    c_ref[...] = jax.lax.dot_general(
        eat_ref[...], w3_ref[...], (((0,), (0,)), ((), ())),
        preferred_element_type=jnp.float32)


def _compute_c(eat, w3):
    BR = 12800
    return pl.pallas_call(
        _c_body,
        grid=(E // BR,),
        in_specs=[pl.BlockSpec((DE, BR), lambda i: (0, i)),
                  pl.BlockSpec((DE, D), lambda i: (0, 0))],
        out_specs=pl.BlockSpec((BR, D), lambda i: (i, 0)),
        out_shape=jax.ShapeDtypeStruct((E, D), jnp.float32),
    )(eat, w3)


def _sc_edge(src, dst, a_t, b_t, c_t, zeros):
    mesh = plsc.VectorSubcoreMesh(core_axis_name="c", subcore_axis_name="s")
    SETS = 3
    ISLOTS = 6
    BODY = 10

    @functools.partial(
        pl.kernel,
        mesh=mesh,
        out_type=jax.ShapeDtypeStruct((NC, NPAD, D), jnp.float32),
        scratch_types=(
            [pltpu.VMEM((K,), jnp.int32) for _ in range(2 * ISLOTS)]
            + [pltpu.VMEM((K, D), jnp.float32) for _ in range(3 * SETS)]
            + [pltpu.VMEM_SHARED((NPAD, D), jnp.float32)]
            + [pltpu.SemaphoreType.DMA for _ in range(6)]
        ),
    )
    def run(src_h, dst_h, a_h, b_h, c_h, z_h, out_h, *refs):
        sidx = refs[0:ISLOTS]
        didx = refs[ISLOTS:2 * ISLOTS]
        av = refs[2 * ISLOTS:2 * ISLOTS + SETS]
        bv = refs[2 * ISLOTS + SETS:2 * ISLOTS + 2 * SETS]
        cv = refs[2 * ISLOTS + 2 * SETS:2 * ISLOTS + 3 * SETS]
        agg = refs[2 * ISLOTS + 3 * SETS]
        sems = refs[2 * ISLOTS + 3 * SETS + 1:]
        isem = sems[0:2]
        gsem = sems[2:5]
        ssem = sems[5]
        cid = lax.axis_index("c")
        sid = lax.axis_index("s")
        wid = cid * NS + sid
        pltpu.sync_copy(z_h, agg.at[pl.ds(sid * RPT, RPT)])
        plsc.subcore_barrier()
        ebase = wid * EPT

        def fetch_idx(n, q):
            off = ebase + n * K
            sl = q % ISLOTS
            pltpu.async_copy(src_h.at[pl.ds(off, K)], sidx[sl], isem[q % 2])
            pltpu.async_copy(dst_h.at[pl.ds(off, K)], didx[sl], isem[q % 2])

        def wait_idx(n, q):
            off = ebase + n * K
            sl = q % ISLOTS
            pltpu.make_async_copy(src_h.at[pl.ds(off, K)], sidx[sl],
                                  isem[q % 2]).wait()
            pltpu.make_async_copy(dst_h.at[pl.ds(off, K)], didx[sl],
                                  isem[q % 2]).wait()

        def fetch_rows(n, q):
            sl, st = q % ISLOTS, q % SETS
            pltpu.async_copy(a_h.at[sidx[sl]], av[st], gsem[q % 3])
            pltpu.async_copy(b_h.at[didx[sl]], bv[st], gsem[q % 3])
            pltpu.async_copy(c_h.at[pl.ds(ebase + n * K, K)], cv[st],
                             gsem[q % 3])

        def wait_rows(n, q):
            sl, st = q % ISLOTS, q % SETS
            pltpu.make_async_copy(a_h.at[sidx[sl]], av[st],
                                  gsem[q % 3]).wait()
            pltpu.make_async_copy(b_h.at[didx[sl]], bv[st],
                                  gsem[q % 3]).wait()
            pltpu.make_async_copy(c_h.at[pl.ds(ebase + n * K, K)], cv[st],
                                  gsem[q % 3]).wait()

        def compute(q):
            st = q % SETS

            def row(i, c2):
                for dr in range(2):
                    r = 2 * i + dr
                    for j in range(8):
                        sl = pl.ds(j * L, L)
                        t = av[st][r, sl] + bv[st][r, sl] + cv[st][r, sl]
                        cv[st][r, sl] = t / (1.0 + jnp.exp(-t))
                return c2
            lax.fori_loop(0, K // 2, row, 0)

        def scat(q):
            pltpu.async_copy(cv[q % SETS], agg.at[didx[q % ISLOTS]], ssem,
                             add=True)

        def wait_scat(q):
            pltpu.make_async_copy(cv[q % SETS], agg.at[didx[q % ISLOTS]],
                                  ssem).wait()

        def body(g, carry):
            base = BODY * g
            fetch_idx(base + 0, 0)
            fetch_idx(base + 1, 1)
            wait_idx(base + 0, 0)
            fetch_rows(base + 0, 0)
            fetch_idx(base + 2, 2)
            wait_idx(base + 1, 1)
            fetch_rows(base + 1, 1)
            fetch_idx(base + 3, 3)
            wait_idx(base + 2, 2)
            fetch_rows(base + 2, 2)
            for q in range(BODY):
                wait_rows(base + q, q)
                compute(q)
                scat(q)
                if q + 4 < BODY:
                    fetch_idx(base + q + 4, q + 4)
                wait_scat(q)
                if q + SETS < BODY:
                    wait_idx(base + q + SETS, q + SETS)
                    fetch_rows(base + q + SETS, q + SETS)
            return carry

        lax.fori_loop(0, NCH // BODY, body, 0)
        plsc.subcore_barrier()
        pltpu.sync_copy(agg.at[pl.ds(sid * RPT, RPT)],
                        out_h.at[cid, pl.ds(sid * RPT, RPT)])

    return run(src, dst, a_t, b_t, c_t, zeros)


def _node_body(x_ref, p_ref, w1_ref, w2_ref, bn_ref, o_ref):
    xb = x_ref[...]
    agg = p_ref[0] + p_ref[1]
    h = (jnp.dot(xb, w1_ref[...], preferred_element_type=jnp.float32)
         + jnp.dot(agg, w2_ref[...], preferred_element_type=jnp.float32)
         + bn_ref[...])
    o_ref[...] = xb + h / (1.0 + jnp.exp(-h))


def _node_update(x, partials, wn1, wn2, bn):
    BR = 1000
    return pl.pallas_call(
        _node_body,
        grid=(N // BR,),
        in_specs=[
            pl.BlockSpec((BR, D), lambda i: (i, 0)),
            pl.BlockSpec((NC, BR, D), lambda i: (0, i, 0)),
            pl.BlockSpec((D, D), lambda i: (0, 0)),
            pl.BlockSpec((D, D), lambda i: (0, 0)),
            pl.BlockSpec((1, D), lambda i: (0, 0)),
        ],
        out_specs=pl.BlockSpec((BR, D), lambda i: (i, 0)),
        out_shape=jax.ShapeDtypeStruct((N, D), jnp.float32),
    )(x, partials, wn1, wn2, bn)


def kernel(x, edge_index, edge_attr, W_edge, b_edge, W_node, b_node):
    src = edge_index[0].astype(jnp.int32)
    dst = edge_index[1].astype(jnp.int32)
    w1 = W_edge[:D]
    w2 = W_edge[D:2 * D]
    w3 = W_edge[2 * D:]
    a_t, b_t = _compute_ab(x, w1, w2, b_edge.reshape(1, D))
    c_t = _compute_c(edge_attr.T, w3)
    zeros = jnp.zeros((RPT, D), jnp.float32)
    partials = _sc_edge(src, dst, a_t, b_t, c_t, zeros)
    return _node_update(x, partials, W_node[:D], W_node[D:], b_node.reshape(1, D))


# speedup vs baseline: 2.1123x; 1.0944x over previous
import functools

import jax
import jax.numpy as jnp
from jax import lax
from jax.experimental import pallas as pl
from jax.experimental.pallas import tpu as pltpu
from jax.experimental.pallas import tpu_sc as plsc

N = 10000
E = 320000
D = 128
DE = 16

NC, NS, L = 2, 16, 16
NW = NC * NS
EPT = E // NW
K = 40
NCH = EPT // K
NPAD = 10112
RPT = NPAD // NS


def _ab_body(x_ref, w1_ref, w2_ref, be_ref, a_ref, b_ref):
    xb = x_ref[...]
    a_ref[...] = jnp.dot(xb, w1_ref[...], preferred_element_type=jnp.float32)
    b_ref[...] = (jnp.dot(xb, w2_ref[...], preferred_element_type=jnp.float32)
                  + be_ref[...])


def _compute_ab(x, w1, w2, be):
    BR = 1000
    return pl.pallas_call(
        _ab_body,
        grid=(N // BR,),
        in_specs=[
            pl.BlockSpec((BR, D), lambda i: (i, 0)),
            pl.BlockSpec((D, D), lambda i: (0, 0)),
            pl.BlockSpec((D, D), lambda i: (0, 0)),
            pl.BlockSpec((1, D), lambda i: (0, 0)),
        ],
        out_specs=[pl.BlockSpec((BR, D), lambda i: (i, 0)),
                   pl.BlockSpec((BR, D), lambda i: (i, 0))],
        out_shape=[jax.ShapeDtypeStruct((N, D), jnp.float32),
                   jax.ShapeDtypeStruct((N, D), jnp.float32)],
    )(x, w1, w2, be)


def _c_body(eat_ref, w3_ref, c_ref):
    c_ref[...] = jax.lax.dot_general(
        eat_ref[...], w3_ref[...], (((0,), (0,)), ((), ())),
        preferred_element_type=jnp.float32)


def _compute_c(eat, w3):
    BR = 12800
    return pl.pallas_call(
        _c_body,
        grid=(E // BR,),
        in_specs=[pl.BlockSpec((DE, BR), lambda i: (0, i)),
                  pl.BlockSpec((DE, D), lambda i: (0, 0))],
        out_specs=pl.BlockSpec((BR, D), lambda i: (i, 0)),
        out_shape=jax.ShapeDtypeStruct((E, D), jnp.float32),
    )(eat, w3)


def _sc_edge(src, dst, a_t, b_t, c_t, zeros):
    mesh = plsc.VectorSubcoreMesh(core_axis_name="c", subcore_axis_name="s")
    SETS = 3
    ISLOTS = 6
    BODY = 25

    @functools.partial(
        pl.kernel,
        mesh=mesh,
        out_type=jax.ShapeDtypeStruct((NC, NPAD, D), jnp.float32),
        scratch_types=(
            [pltpu.VMEM((K,), jnp.int32) for _ in range(2 * ISLOTS)]
            + [pltpu.VMEM((K, D), jnp.float32) for _ in range(3 * SETS)]
            + [pltpu.VMEM_SHARED((NPAD, D), jnp.float32)]
            + [pltpu.SemaphoreType.DMA for _ in range(6)]
        ),
    )
    def run(src_h, dst_h, a_h, b_h, c_h, z_h, out_h, *refs):
        sidx = refs[0:ISLOTS]
        didx = refs[ISLOTS:2 * ISLOTS]
        av = refs[2 * ISLOTS:2 * ISLOTS + SETS]
        bv = refs[2 * ISLOTS + SETS:2 * ISLOTS + 2 * SETS]
        cv = refs[2 * ISLOTS + 2 * SETS:2 * ISLOTS + 3 * SETS]
        agg = refs[2 * ISLOTS + 3 * SETS]
        sems = refs[2 * ISLOTS + 3 * SETS + 1:]
        isem = sems[0:2]
        gsem = sems[2:5]
        ssem = sems[5]
        cid = lax.axis_index("c")
        sid = lax.axis_index("s")
        wid = cid * NS + sid
        pltpu.sync_copy(z_h, agg.at[pl.ds(sid * RPT, RPT)])
        plsc.subcore_barrier()
        ebase = wid * EPT

        def fetch_idx(n, q):
            off = ebase + n * K
            sl = q % ISLOTS
            pltpu.async_copy(src_h.at[pl.ds(off, K)], sidx[sl], isem[q % 2])
            pltpu.async_copy(dst_h.at[pl.ds(off, K)], didx[sl], isem[q % 2])

        def wait_idx(n, q):
            off = ebase + n * K
            sl = q % ISLOTS
            pltpu.make_async_copy(src_h.at[pl.ds(off, K)], sidx[sl],
                                  isem[q % 2]).wait()
            pltpu.make_async_copy(dst_h.at[pl.ds(off, K)], didx[sl],
                                  isem[q % 2]).wait()

        def fetch_rows(n, q):
            sl, st = q % ISLOTS, q % SETS
            pltpu.async_copy(a_h.at[sidx[sl]], av[st], gsem[q % 3])
            pltpu.async_copy(b_h.at[didx[sl]], bv[st], gsem[q % 3])
            pltpu.async_copy(c_h.at[pl.ds(ebase + n * K, K)], cv[st],
                             gsem[q % 3])

        def wait_rows(n, q):
            sl, st = q % ISLOTS, q % SETS
            pltpu.make_async_copy(a_h.at[sidx[sl]], av[st],
                                  gsem[q % 3]).wait()
            pltpu.make_async_copy(b_h.at[didx[sl]], bv[st],
                                  gsem[q % 3]).wait()
            pltpu.make_async_copy(c_h.at[pl.ds(ebase + n * K, K)], cv[st],
                                  gsem[q % 3]).wait()

        def compute(q):
            st = q % SETS

            def row(i, c2):
                for dr in range(2):
                    r = 2 * i + dr
                    for j in range(8):
                        sl = pl.ds(j * L, L)
                        t = av[st][r, sl] + bv[st][r, sl] + cv[st][r, sl]
                        cv[st][r, sl] = t / (1.0 + jnp.exp(-t))
                return c2
            lax.fori_loop(0, K // 2, row, 0)

        def scat(q):
            pltpu.async_copy(cv[q % SETS], agg.at[didx[q % ISLOTS]], ssem,
                             add=True)

        def wait_scat(q):
            pltpu.make_async_copy(cv[q % SETS], agg.at[didx[q % ISLOTS]],
                                  ssem).wait()

        def body(g, carry):
            base = BODY * g
            fetch_idx(base + 0, 0)
            fetch_idx(base + 1, 1)
            wait_idx(base + 0, 0)
            fetch_rows(base + 0, 0)
            fetch_idx(base + 2, 2)
            wait_idx(base + 1, 1)
            fetch_rows(base + 1, 1)
            fetch_idx(base + 3, 3)
            wait_idx(base + 2, 2)
            fetch_rows(base + 2, 2)
            for q in range(BODY):
                wait_rows(base + q, q)
                compute(q)
                scat(q)
                if q + 4 < BODY:
                    fetch_idx(base + q + 4, q + 4)
                wait_scat(q)
                if q + SETS < BODY:
                    wait_idx(base + q + SETS, q + SETS)
                    fetch_rows(base + q + SETS, q + SETS)
            return carry

        lax.fori_loop(0, NCH // BODY, body, 0)
        plsc.subcore_barrier()
        pltpu.sync_copy(agg.at[pl.ds(sid * RPT, RPT)],
                        out_h.at[cid, pl.ds(sid * RPT, RPT)])

    return run(src, dst, a_t, b_t, c_t, zeros)


def _node_body(x_ref, p_ref, w1_ref, w2_ref, bn_ref, o_ref):
    xb = x_ref[...]
    agg = p_ref[0] + p_ref[1]
    h = (jnp.dot(xb, w1_ref[...], preferred_element_type=jnp.float32)
         + jnp.dot(agg, w2_ref[...], preferred_element_type=jnp.float32)
         + bn_ref[...])
    o_ref[...] = xb + h / (1.0 + jnp.exp(-h))


def _node_update(x, partials, wn1, wn2, bn):
    BR = 1000
    return pl.pallas_call(
        _node_body,
        grid=(N // BR,),
        in_specs=[
            pl.BlockSpec((BR, D), lambda i: (i, 0)),
            pl.BlockSpec((NC, BR, D), lambda i: (0, i, 0)),
            pl.BlockSpec((D, D), lambda i: (0, 0)),
            pl.BlockSpec((D, D), lambda i: (0, 0)),
            pl.BlockSpec((1, D), lambda i: (0, 0)),
        ],
        out_specs=pl.BlockSpec((BR, D), lambda i: (i, 0)),
        out_shape=jax.ShapeDtypeStruct((N, D), jnp.float32),
    )(x, partials, wn1, wn2, bn)


def kernel(x, edge_index, edge_attr, W_edge, b_edge, W_node, b_node):
    src = edge_index[0].astype(jnp.int32)
    dst = edge_index[1].astype(jnp.int32)
    w1 = W_edge[:D]
    w2 = W_edge[D:2 * D]
    w3 = W_edge[2 * D:]
    a_t, b_t = _compute_ab(x, w1, w2, b_edge.reshape(1, D))
    c_t = _compute_c(edge_attr.T, w3)
    zeros = jnp.zeros((RPT, D), jnp.float32)
    partials = _sc_edge(src, dst, a_t, b_t, c_t, zeros)
    return _node_update(x, partials, W_node[:D], W_node[D:], b_node.reshape(1, D))
